# baseline (speedup 1.0000x reference)
import functools

import jax
import jax.numpy as jnp
from jax import lax
from jax.experimental import pallas as pl
from jax.experimental.pallas import tpu as pltpu
from jax.experimental.pallas import tpu_sc as plsc

NV = 40962
NV_PREV = 10242
NF = 81920
BS = 2
C = 128
D = BS * C

_NC, _NSC = 2, 16
NW = _NC * _NSC
NVP = 41472
NVPL = 43008
VPW = NVPL // NW

FCH = 16
FNCH = (NF // NW) // FCH
LCH = 16
LNCH = VPW // LCH
VCH = 16
VNCH = VPW // VCH

_MESH = dict(core_axis_name="c", subcore_axis_name="s",
             num_cores=_NC, num_subcores=_NSC)


def _wid():
    return lax.axis_index("s") * _NC + lax.axis_index("c")


def _emit_pass(idx_src, w_src, table, out_hbm, idxb, wbs, gb, ob, sems,
               rch, nch, compute_rows):
    smi = sems[0:2]
    smw = sems[2:4]
    smg = sems[4:6]
    smo = sems[6:8]
    c0 = _wid() * nch

    def issue_idx(c, s, sem_slot):
        for src, dst in zip(idx_src(c), idxb[s]):
            pltpu.async_copy(src, dst, smi[sem_slot])

    def wait_idx(c, s, sem_slot):
        for src, dst in zip(idx_src(c), idxb[s]):
            pltpu.make_async_copy(src, dst, smi[sem_slot]).wait()

    def issue_w(c, s):
        for src, dst in zip(w_src(c), wbs[s]):
            pltpu.async_copy(src, dst, smw[s])

    def wait_w(c, s):
        for src, dst in zip(w_src(c), wbs[s]):
            pltpu.make_async_copy(src, dst, smw[s]).wait()

    def issue_gather(s):
        for idxv, gdst in zip(idxb[s], gb[s]):
            pltpu.async_copy(table.at[idxv], gdst, smg[s])

    def wait_gather(s):
        for idxv, gdst in zip(idxb[s], gb[s]):
            pltpu.make_async_copy(table.at[idxv], gdst, smg[s]).wait()

    issue_idx(c0, 0, 0)
    wait_idx(c0, 0, 0)
    issue_idx(c0 + 1, 1, 1)
    issue_w(c0, 0)
    issue_gather(0)

    def pair(p, carry):
        for b in range(2):
            ci = 2 * p + b
            c = c0 + ci
            s, s1 = b, 1 - b
            wait_gather(s)

            @pl.when(ci + 2 < nch)
            def _():
                issue_idx(c + 2, s, s)

            @pl.when(ci + 1 < nch)
            def _():
                wait_idx(c + 1, s1, s1)
                issue_w(c + 1, s1)
                issue_gather(s1)

            wait_w(c, s)

            @pl.when(ci >= 2)
            def _():
                pltpu.make_async_copy(
                    ob[s], out_hbm.at[pl.ds(c * rch, rch)], smo[s]).wait()

            compute_rows(gb[s], wbs[s], ob[s])
            pltpu.async_copy(ob[s], out_hbm.at[pl.ds(c * rch, rch)], smo[s])
        return carry

    lax.fori_loop(0, nch // 2, pair, 0)
    pltpu.make_async_copy(ob[0], out_hbm.at[pl.ds(c0 * rch, rch)], smo[0]).wait()
    pltpu.make_async_copy(ob[1], out_hbm.at[pl.ds(c0 * rch, rch)], smo[1]).wait()


def _face_rows(gbs, wvs, ob):
    g0b, g1b, g2b = gbs
    w0b, w1b, w2b, ewb, nsb = wvs

    @plsc.parallel_loop(0, FCH)
    def face(fb):
        g0 = fb * 3
        wv0 = w0b[pl.ds(g0, 16)]
        wv1 = w1b[pl.ds(g0, 16)]
        wv2 = w2b[pl.ds(g0, 16)]
        ev = ewb[pl.ds(g0, 16)]
        nv = nsb[pl.ds(g0, 16)]
        for t in range(D // 16):
            sl = pl.ds(t * 16, 16)
            s0 = wv0[0] * g0b[g0, sl] + wv0[1] * g0b[g0 + 1, sl] + wv0[2] * g0b[g0 + 2, sl]
            s1 = wv1[0] * g1b[g0, sl] + wv1[1] * g1b[g0 + 1, sl] + wv1[2] * g1b[g0 + 2, sl]
            s2 = wv2[0] * g2b[g0, sl] + wv2[1] * g2b[g0 + 1, sl] + wv2[2] * g2b[g0 + 2, sl]
            ob[fb, sl] = ev[0] * s0 + ev[1] * s1 + ev[2] * s2
            ob[fb, pl.ds(D + t * 16, 16)] = nv[0] * s0 + nv[1] * s1 + nv[2] * s2


def _lap_rows(gbs, wvs, ob):
    wvb = wvs[0]
    for h, gb in enumerate(gbs):
        @plsc.parallel_loop(0, LCH // 2)
        def vert(vbl, _h=h, _gb=gb):
            g0 = vbl * 7
            wv = wvb[pl.ds(_h * 56 + g0, 16)]
            w = [wv[j] for j in range(7)]
            for t in range(D // 16):
                sl = pl.ds(t * 16, 16)
                acc = w[0] * _gb[g0 + 0, sl]
                for j in range(1, 7):
                    acc = acc + w[j] * _gb[g0 + j, sl]
                ob[_h * (LCH // 2) + vbl, sl] = acc


def _f2v_rows(gbs, wvs, ob):
    wvb = wvs[0]
    for h, gb in enumerate(gbs):
        @plsc.parallel_loop(0, VCH // 2)
        def vert(vbl, _h=h, _gb=gb):
            g0 = vbl * 6
            wv = wvb[pl.ds(_h * 48 + g0, 16)]
            w = [wv[j] for j in range(6)]
            for t in range(2 * D // 16):
                sl = pl.ds(t * 16, 16)
                acc = w[0] * _gb[g0 + 0, sl]
                for j in range(1, 6):
                    acc = acc + w[j] * _gb[g0 + j, sl]
                ob[_h * (VCH // 2) + vbl, sl] = acc


def _face_body(xt, gcols, gvals, ewf, nsf, faces2, *scr):
    idxb = (scr[0:3], scr[3:6])
    wbs = (scr[6:11], scr[11:16])
    gb = (scr[16:19], scr[19:22])
    ob = scr[22:24]
    sems = scr[24:32]

    def idx_src(c):
        return [gcols.at[pl.ds(d * 3 * NF + c * 48, 48)] for d in range(3)]

    def w_src(c):
        return ([gvals.at[pl.ds(d * 3 * NF + c * 48, 64)] for d in range(3)]
                + [ewf.at[pl.ds(c * 48, 64)], nsf.at[pl.ds(c * 48, 64)]])

    _emit_pass(idx_src, w_src, xt, faces2, idxb, wbs, gb, ob, sems,
               FCH, FNCH, _face_rows)


def _sc_face(xt, gcols, gvals, ewf, nsf):
    scr = (
        [pltpu.VMEM((48,), jnp.int32)] * 6
        + [pltpu.VMEM((64,), jnp.float32)] * 10
        + [pltpu.VMEM((48, D), jnp.float32)] * 6
        + [pltpu.VMEM((FCH, 2 * D), jnp.float32)] * 2
        + [pltpu.SemaphoreType.DMA] * 8
    )
    return pl.kernel(
        _face_body,
        out_type=jax.ShapeDtypeStruct((NF, 2 * D), jnp.float32),
        mesh=plsc.VectorSubcoreMesh(**_MESH),
        scratch_types=scr,
    )(xt, gcols, gvals, ewf, nsf)


def _vert_body_maker(k, rch, nch, rows_fn, wfetch):
    half = (rch // 2) * k

    def body(table, cols, vals, out, *scr):
        idxb = (scr[0:2], scr[2:4])
        wbs = (scr[4:5], scr[5:6])
        gb = (scr[6:8], scr[8:10])
        ob = scr[10:12]
        sems = scr[12:20]

        def idx_src(c):
            return [cols.at[pl.ds(c * (rch * k) + h * half, half)]
                    for h in range(2)]

        def w_src(c):
            return [vals.at[pl.ds(c * (rch * k), wfetch)]]

        _emit_pass(idx_src, w_src, table, out, idxb, wbs, gb, ob, sems,
                   rch, nch, rows_fn)

    return body


def _sc_lap(xt, lcols, lvals):
    scr = (
        [pltpu.VMEM((56,), jnp.int32)] * 4
        + [pltpu.VMEM((128,), jnp.float32)] * 2
        + [pltpu.VMEM((56, D), jnp.float32)] * 4
        + [pltpu.VMEM((LCH, D), jnp.float32)] * 2
        + [pltpu.SemaphoreType.DMA] * 8
    )
    return pl.kernel(
        _vert_body_maker(7, LCH, LNCH, _lap_rows, 128),
        out_type=jax.ShapeDtypeStruct((NVPL, D), jnp.float32),
        mesh=plsc.VectorSubcoreMesh(**_MESH),
        scratch_types=scr,
    )(xt, lcols, lvals)


def _sc_f2v(faces2, fcols, fvals):
    scr = (
        [pltpu.VMEM((48,), jnp.int32)] * 4
        + [pltpu.VMEM((112,), jnp.float32)] * 2
        + [pltpu.VMEM((48, 2 * D), jnp.float32)] * 4
        + [pltpu.VMEM((VCH, 2 * D), jnp.float32)] * 2
        + [pltpu.SemaphoreType.DMA] * 8
    )
    return pl.kernel(
        _vert_body_maker(6, VCH, VNCH, _f2v_rows, 112),
        out_type=jax.ShapeDtypeStruct((NVPL, 2 * D), jnp.float32),
        mesh=plsc.VectorSubcoreMesh(**_MESH),
        scratch_types=scr,
    )(faces2, fcols, fvals)


_BLK = 512


def _xtp_body(xp_ref, out_ref):
    i = pl.program_id(0)
    base = jnp.minimum(i, (NV_PREV // _BLK)) * _BLK
    valid = (i * _BLK + lax.broadcasted_iota(jnp.int32, (_BLK, C), 0)) < NV_PREV
    for b in range(BS):
        vals = xp_ref[b, :, pl.ds(base, _BLK)].T
        out_ref[:, b * C:(b + 1) * C] = jnp.where(valid, vals, 1.0)


def _tc_xtp(xpad):
    return pl.pallas_call(
        _xtp_body,
        grid=(NVP // _BLK,),
        in_specs=[pl.BlockSpec((BS, C, NV_PREV + (_BLK - NV_PREV % _BLK)),
                               lambda i: (0, 0, 0))],
        out_specs=pl.BlockSpec((_BLK, D), lambda i: (i, 0)),
        out_shape=jax.ShapeDtypeStruct((NVP, D), jnp.float32),
    )(xpad)


def _combine1_body(xt_ref, lap_ref, cs_ref, out_ref):
    cs = cs_ref[...]
    for b in range(BS):
        x = xt_ref[:, b * C:(b + 1) * C]
        l = lap_ref[:, b * C:(b + 1) * C]
        acc = (jnp.dot(x, cs[0:C], preferred_element_type=jnp.float32)
               + jnp.dot(l, cs[C:2 * C], preferred_element_type=jnp.float32))
        out_ref[b] = acc.T


def _combine2_body(part_ref, gv_ref, cs_ref, out_ref):
    cs = cs_ref[...]
    for b in range(BS):
        e = gv_ref[:, b * C:(b + 1) * C]
        n = gv_ref[:, D + b * C:D + (b + 1) * C]
        acc = (jnp.dot(e, cs[0:C], preferred_element_type=jnp.float32)
               + jnp.dot(n, cs[C:2 * C], preferred_element_type=jnp.float32))
        out_ref[b] = part_ref[b] + acc.T


def _tc_combine1(xtp, lap, cs01):
    return pl.pallas_call(
        _combine1_body,
        grid=(NVP // _BLK,),
        in_specs=[
            pl.BlockSpec((_BLK, D), lambda i: (i, 0)),
            pl.BlockSpec((_BLK, D), lambda i: (i, 0)),
            pl.BlockSpec((2 * C, C), lambda i: (0, 0)),
        ],
        out_specs=pl.BlockSpec((BS, C, _BLK), lambda i: (0, 0, i)),
        out_shape=jax.ShapeDtypeStruct((BS, C, NVP), jnp.float32),
    )(xtp, lap, cs01)


def _tc_combine2(part, gv, cs23):
    return pl.pallas_call(
        _combine2_body,
        grid=(NVP // _BLK,),
        in_specs=[
            pl.BlockSpec((BS, C, _BLK), lambda i: (0, 0, i)),
            pl.BlockSpec((_BLK, 2 * D), lambda i: (i, 0)),
            pl.BlockSpec((2 * C, C), lambda i: (0, 0)),
        ],
        out_specs=pl.BlockSpec((BS, C, _BLK), lambda i: (0, 0, i)),
        out_shape=jax.ShapeDtypeStruct((BS, C, NV), jnp.float32),
    )(part, gv, cs23)


def _pad1d(a, n, dtype):
    return jnp.concatenate([a.reshape(-1), jnp.zeros((n - a.size,), dtype)])


def kernel(input, coeffs, G_rows, G_cols, G_vals, L_rows, L_cols, L_vals,
           F_rows, F_cols, F_vals, NS, EW):
    f32 = jnp.float32
    i32 = jnp.int32
    xpad = jnp.concatenate(
        [input, jnp.zeros((BS, C, 510), dtype=input.dtype)], axis=-1)
    xtp = _tc_xtp(xpad)

    gvalsf = _pad1d(G_vals, 3 * 3 * NF + 64, f32)
    ewf = _pad1d(EW, 3 * NF + 64, f32)
    nsf = _pad1d(NS, 3 * NF + 64, f32)

    lcols = _pad1d(L_cols, NVPL * 7, i32)
    lvals = _pad1d(L_vals, NVPL * 7 + 128, f32)
    fcols = _pad1d(F_cols, NVPL * 6, i32)
    fvals = _pad1d(F_vals, NVPL * 6 + 112, f32)

    cstack = coeffs.reshape(C, 4, C).transpose(1, 0, 2).reshape(4 * C, C)
    cs01 = cstack[0:2 * C]
    cs23 = cstack[2 * C:4 * C]

    faces2 = _sc_face(xtp, G_cols, gvalsf, ewf, nsf)
    lap = _sc_lap(xtp, lcols, lvals)
    part = _tc_combine1(xtp, lap, cs01)
    gv = _sc_f2v(faces2, fcols, fvals)
    return _tc_combine2(part, gv, cs23)


# speedup vs baseline: 1.4626x; 1.4626x over previous
import functools

import jax
import jax.numpy as jnp
from jax import lax
from jax.experimental import pallas as pl
from jax.experimental.pallas import tpu as pltpu
from jax.experimental.pallas import tpu_sc as plsc

NV = 40962
NV_PREV = 10242
NF = 81920
BS = 2
C = 128
D = BS * C

_NC, _NSC = 2, 16
NW = _NC * _NSC
NVP = 41472
VPW = NVP // NW

FCH = 16
FNCH = (NF // NW) // FCH
LCH = 8
LNCH = VPW // LCH
VCH = 8
VNCH = VPW // VCH

_MESH = dict(core_axis_name="c", subcore_axis_name="s",
             num_cores=_NC, num_subcores=_NSC)


def _wid():
    return lax.axis_index("s") * _NC + lax.axis_index("c")


def _emit_pass(idx_src, w_src, table, out_hbm, idxb, wbs, gb, ob, sems,
               rch, nch, compute_rows):
    smi = sems[0:2]
    smw = sems[2:4]
    smg = sems[4:6]
    smo = sems[6:8]
    c0 = _wid() * nch

    def issue_idx(c, s, sem_slot):
        for src, dst in zip(idx_src(c), idxb[s]):
            pltpu.async_copy(src, dst, smi[sem_slot])

    def wait_idx(c, s, sem_slot):
        for src, dst in zip(idx_src(c), idxb[s]):
            pltpu.make_async_copy(src, dst, smi[sem_slot]).wait()

    def issue_w(c, s):
        for src, dst in zip(w_src(c), wbs[s]):
            pltpu.async_copy(src, dst, smw[s])

    def wait_w(c, s):
        for src, dst in zip(w_src(c), wbs[s]):
            pltpu.make_async_copy(src, dst, smw[s]).wait()

    def issue_gather(s):
        for idxv, gdst in zip(idxb[s], gb[s]):
            pltpu.async_copy(table.at[idxv], gdst, smg[s])

    def wait_gather(s):
        for idxv, gdst in zip(idxb[s], gb[s]):
            pltpu.make_async_copy(table.at[idxv], gdst, smg[s]).wait()

    issue_idx(c0, 0, 0)
    wait_idx(c0, 0, 0)
    issue_idx(c0 + 1, 1, 1)
    issue_w(c0, 0)
    issue_gather(0)

    def pair(p, carry):
        for b in range(2):
            ci = 2 * p + b
            c = c0 + ci
            s, s1 = b, 1 - b
            wait_gather(s)

            @pl.when(ci + 2 < nch)
            def _():
                issue_idx(c + 2, s, s)

            @pl.when(ci + 1 < nch)
            def _():
                wait_idx(c + 1, s1, s1)
                issue_w(c + 1, s1)
                issue_gather(s1)

            wait_w(c, s)

            @pl.when(ci >= 2)
            def _():
                pltpu.make_async_copy(
                    ob[s], out_hbm.at[pl.ds(c * rch, rch)], smo[s]).wait()

            compute_rows(gb[s], wbs[s], ob[s])
            pltpu.async_copy(ob[s], out_hbm.at[pl.ds(c * rch, rch)], smo[s])
        return carry

    lax.fori_loop(0, nch // 2, pair, 0)
    pltpu.make_async_copy(ob[0], out_hbm.at[pl.ds(c0 * rch, rch)], smo[0]).wait()
    pltpu.make_async_copy(ob[1], out_hbm.at[pl.ds(c0 * rch, rch)], smo[1]).wait()


def _face_rows(gbs, wvs, ob):
    g0b, g1b, g2b = gbs
    w0b, w1b, w2b, ewb, nsb = wvs

    @plsc.parallel_loop(0, FCH)
    def face(fb):
        g0 = fb * 3
        wv0 = w0b[pl.ds(g0, 16)]
        wv1 = w1b[pl.ds(g0, 16)]
        wv2 = w2b[pl.ds(g0, 16)]
        ev = ewb[pl.ds(g0, 16)]
        nv = nsb[pl.ds(g0, 16)]
        for t in range(D // 16):
            sl = pl.ds(t * 16, 16)
            s0 = wv0[0] * g0b[g0, sl] + wv0[1] * g0b[g0 + 1, sl] + wv0[2] * g0b[g0 + 2, sl]
            s1 = wv1[0] * g1b[g0, sl] + wv1[1] * g1b[g0 + 1, sl] + wv1[2] * g1b[g0 + 2, sl]
            s2 = wv2[0] * g2b[g0, sl] + wv2[1] * g2b[g0 + 1, sl] + wv2[2] * g2b[g0 + 2, sl]
            ob[fb, sl] = ev[0] * s0 + ev[1] * s1 + ev[2] * s2
            ob[fb, pl.ds(D + t * 16, 16)] = nv[0] * s0 + nv[1] * s1 + nv[2] * s2


def _lap_rows(gbs, wvs, ob):
    gb = gbs[0]
    wvb = wvs[0]

    @plsc.parallel_loop(0, LCH)
    def vert(vb):
        g0 = vb * 7
        wv = wvb[pl.ds(g0, 16)]
        w = [wv[j] for j in range(7)]
        for t in range(D // 16):
            sl = pl.ds(t * 16, 16)
            acc = w[0] * gb[g0 + 0, sl]
            for j in range(1, 7):
                acc = acc + w[j] * gb[g0 + j, sl]
            ob[vb, sl] = acc


def _f2v_rows(gbs, wvs, ob):
    gb = gbs[0]
    wvb = wvs[0]

    @plsc.parallel_loop(0, VCH)
    def vert(vb):
        g0 = vb * 6
        wv = wvb[pl.ds(g0, 16)]
        w = [wv[j] for j in range(6)]
        for t in range(2 * D // 16):
            sl = pl.ds(t * 16, 16)
            acc = w[0] * gb[g0 + 0, sl]
            for j in range(1, 6):
                acc = acc + w[j] * gb[g0 + j, sl]
            ob[vb, sl] = acc


def _face_body(xt, gcols, gvals, ewf, nsf, faces2, *scr):
    idxb = (scr[0:3], scr[3:6])
    wbs = (scr[6:11], scr[11:16])
    gb = (scr[16:19], scr[19:22])
    ob = scr[22:24]
    sems = scr[24:32]

    def idx_src(c):
        return [gcols.at[pl.ds(d * 3 * NF + c * 48, 48)] for d in range(3)]

    def w_src(c):
        return ([gvals.at[pl.ds(d * 3 * NF + c * 48, 64)] for d in range(3)]
                + [ewf.at[pl.ds(c * 48, 64)], nsf.at[pl.ds(c * 48, 64)]])

    _emit_pass(idx_src, w_src, xt, faces2, idxb, wbs, gb, ob, sems,
               FCH, FNCH, _face_rows)


def _sc_face(xt, gcols, gvals, ewf, nsf):
    scr = (
        [pltpu.VMEM((48,), jnp.int32)] * 6
        + [pltpu.VMEM((64,), jnp.float32)] * 10
        + [pltpu.VMEM((48, D), jnp.float32)] * 6
        + [pltpu.VMEM((FCH, 2 * D), jnp.float32)] * 2
        + [pltpu.SemaphoreType.DMA] * 8
    )
    return pl.kernel(
        _face_body,
        out_type=jax.ShapeDtypeStruct((NF, 2 * D), jnp.float32),
        mesh=plsc.VectorSubcoreMesh(**_MESH),
        scratch_types=scr,
    )(xt, gcols, gvals, ewf, nsf)


def _vert_body_maker(k, rch, nch, rows_fn, wfetch):
    def body(table, cols, vals, out, *scr):
        idxb = (scr[0:1], scr[1:2])
        wbs = (scr[2:3], scr[3:4])
        gb = (scr[4:5], scr[5:6])
        ob = scr[6:8]
        sems = scr[8:16]

        def idx_src(c):
            return [cols.at[pl.ds(c * (rch * k), rch * k)]]

        def w_src(c):
            return [vals.at[pl.ds(c * (rch * k), wfetch)]]

        _emit_pass(idx_src, w_src, table, out, idxb, wbs, gb, ob, sems,
                   rch, nch, rows_fn)

    return body


def _sc_lap(xt, lcols, lvals):
    scr = (
        [pltpu.VMEM((LCH * 7,), jnp.int32)] * 2
        + [pltpu.VMEM((72,), jnp.float32)] * 2
        + [pltpu.VMEM((LCH * 7, D), jnp.float32)] * 2
        + [pltpu.VMEM((LCH, D), jnp.float32)] * 2
        + [pltpu.SemaphoreType.DMA] * 8
    )
    return pl.kernel(
        _vert_body_maker(7, LCH, LNCH, _lap_rows, 72),
        out_type=jax.ShapeDtypeStruct((NVP, D), jnp.float32),
        mesh=plsc.VectorSubcoreMesh(**_MESH),
        scratch_types=scr,
    )(xt, lcols, lvals)


def _sc_f2v(faces2, fcols, fvals):
    scr = (
        [pltpu.VMEM((VCH * 6,), jnp.int32)] * 2
        + [pltpu.VMEM((64,), jnp.float32)] * 2
        + [pltpu.VMEM((VCH * 6, 2 * D), jnp.float32)] * 2
        + [pltpu.VMEM((VCH, 2 * D), jnp.float32)] * 2
        + [pltpu.SemaphoreType.DMA] * 8
    )
    return pl.kernel(
        _vert_body_maker(6, VCH, VNCH, _f2v_rows, 64),
        out_type=jax.ShapeDtypeStruct((NVP, 2 * D), jnp.float32),
        mesh=plsc.VectorSubcoreMesh(**_MESH),
        scratch_types=scr,
    )(faces2, fcols, fvals)


_BLK = 512


def _xtp_body(xp_ref, out_ref):
    i = pl.program_id(0)
    base = jnp.minimum(i, (NV_PREV // _BLK)) * _BLK
    valid = (i * _BLK + lax.broadcasted_iota(jnp.int32, (_BLK, C), 0)) < NV_PREV
    for b in range(BS):
        vals = xp_ref[b, :, pl.ds(base, _BLK)].T
        out_ref[:, b * C:(b + 1) * C] = jnp.where(valid, vals, 1.0)


def _tc_xtp(xpad):
    return pl.pallas_call(
        _xtp_body,
        grid=(NVP // _BLK,),
        in_specs=[pl.BlockSpec((BS, C, NV_PREV + (_BLK - NV_PREV % _BLK)),
                               lambda i: (0, 0, 0))],
        out_specs=pl.BlockSpec((_BLK, D), lambda i: (i, 0)),
        out_shape=jax.ShapeDtypeStruct((NVP, D), jnp.float32),
    )(xpad)


def _combine1_body(xt_ref, lap_ref, cs_ref, out_ref):
    cs = cs_ref[...]
    for b in range(BS):
        x = xt_ref[:, b * C:(b + 1) * C]
        l = lap_ref[:, b * C:(b + 1) * C]
        acc = (jnp.dot(x, cs[0:C], preferred_element_type=jnp.float32)
               + jnp.dot(l, cs[C:2 * C], preferred_element_type=jnp.float32))
        out_ref[b] = acc.T


def _combine2_body(part_ref, gv_ref, cs_ref, out_ref):
    cs = cs_ref[...]
    for b in range(BS):
        e = gv_ref[:, b * C:(b + 1) * C]
        n = gv_ref[:, D + b * C:D + (b + 1) * C]
        acc = (jnp.dot(e, cs[0:C], preferred_element_type=jnp.float32)
               + jnp.dot(n, cs[C:2 * C], preferred_element_type=jnp.float32))
        out_ref[b] = part_ref[b] + acc.T


def _tc_combine1(xtp, lap, cs01):
    return pl.pallas_call(
        _combine1_body,
        grid=(NVP // _BLK,),
        in_specs=[
            pl.BlockSpec((_BLK, D), lambda i: (i, 0)),
            pl.BlockSpec((_BLK, D), lambda i: (i, 0)),
            pl.BlockSpec((2 * C, C), lambda i: (0, 0)),
        ],
        out_specs=pl.BlockSpec((BS, C, _BLK), lambda i: (0, 0, i)),
        out_shape=jax.ShapeDtypeStruct((BS, C, NVP), jnp.float32),
    )(xtp, lap, cs01)


def _tc_combine2(part, gv, cs23):
    return pl.pallas_call(
        _combine2_body,
        grid=(NVP // _BLK,),
        in_specs=[
            pl.BlockSpec((BS, C, _BLK), lambda i: (0, 0, i)),
            pl.BlockSpec((_BLK, 2 * D), lambda i: (i, 0)),
            pl.BlockSpec((2 * C, C), lambda i: (0, 0)),
        ],
        out_specs=pl.BlockSpec((BS, C, _BLK), lambda i: (0, 0, i)),
        out_shape=jax.ShapeDtypeStruct((BS, C, NV), jnp.float32),
    )(part, gv, cs23)


def _pad1d(a, n, dtype):
    return jnp.concatenate([a.reshape(-1), jnp.zeros((n - a.size,), dtype)])


def kernel(input, coeffs, G_rows, G_cols, G_vals, L_rows, L_cols, L_vals,
           F_rows, F_cols, F_vals, NS, EW):
    f32 = jnp.float32
    i32 = jnp.int32
    xpad = jnp.concatenate(
        [input, jnp.zeros((BS, C, 510), dtype=input.dtype)], axis=-1)
    xtp = _tc_xtp(xpad)

    gvalsf = _pad1d(G_vals, 3 * 3 * NF + 64, f32)
    ewf = _pad1d(EW, 3 * NF + 64, f32)
    nsf = _pad1d(NS, 3 * NF + 64, f32)

    lcols = _pad1d(L_cols, NVP * 7, i32)
    lvals = _pad1d(L_vals, NVP * 7 + 72, f32)
    fcols = _pad1d(F_cols, NVP * 6, i32)
    fvals = _pad1d(F_vals, NVP * 6 + 64, f32)

    cstack = coeffs.reshape(C, 4, C).transpose(1, 0, 2).reshape(4 * C, C)
    cs01 = cstack[0:2 * C]
    cs23 = cstack[2 * C:4 * C]

    faces2 = _sc_face(xtp, G_cols, gvalsf, ewf, nsf)
    lap = _sc_lap(xtp, lcols, lvals)
    part = _tc_combine1(xtp, lap, cs01)
    gv = _sc_f2v(faces2, fcols, fvals)
    return _tc_combine2(part, gv, cs23)
